# Initial kernel scaffold; baseline (speedup 1.0000x reference)
import jax
import jax.numpy as jnp
from jax.experimental import pallas as pl


def kernel(origins, directions, nears, fars):
    raise NotImplementedError("write your pallas kernel here")


# speedup vs baseline: 6.6307x; 6.6307x over previous
import functools

import jax
import jax.numpy as jnp
from jax import lax
from jax.experimental import pallas as pl
from jax.experimental.pallas import tpu as pltpu
from jax.experimental.pallas import tpu_sc as plsc

NSAMP = 64
NSTEP = 4
NPER = 16
NBINS = NPER + 1
BASEVAR = 64.0
NRAYS = 65536
L = 16
NC, NS = 2, 16
NWORK = NC * NS
GROUPS = NRAYS // L
GPW = GROUPS // NWORK
CH = 8
NCHUNK = GPW // CH
SOUT = NSAMP + NSTEP * NPER
NOUT = SOUT + 1


def _sqrtv(a):
    i = lax.bitcast_convert_type(a, jnp.int32)
    x = lax.bitcast_convert_type(jnp.int32(0x5F3759DF) - (i >> 1), jnp.float32)
    for _ in range(3):
        x = x * (1.5 - 0.5 * a * x * x)
    return a * x


def _sigmoid(z):
    return 1.0 / (1.0 + jnp.exp(-z))


def _search_right(ref, a0, v, length, lane):
    pos = jnp.zeros((L,), jnp.int32)
    step = 64
    while step >= 1:
        cand = jnp.minimum(pos + step, length - 1)
        av = plsc.load_gather(ref, [cand * L + lane])
        pos = jnp.where(av <= v, cand, pos)
        step //= 2
    return jnp.where(a0 <= v, pos + 1, 0)


def _sc_body(inp_hbm, out_hbm, inp_v, outc_v,
             spb_a, sdf_a, spb_b, sdf_b, wbuf, cdf, nbuf, sdfb, delta):
    cid = lax.axis_index("c")
    sid = lax.axis_index("s")
    wid = sid * NC + cid
    lane = lax.iota(jnp.int32, L)
    zf = jnp.zeros((L,), jnp.float32)
    zi = jnp.zeros((L,), jnp.int32)
    onei = jnp.ones((L,), jnp.int32)

    pltpu.sync_copy(inp_hbm.at[pl.ds(wid * (GPW * 8 * L), GPW * 8 * L)], inp_v)

    def row(ref, i):
        return ref[pl.ds(i * L, L)]

    def setrow(ref, i, v):
        ref[pl.ds(i * L, L)] = v

    def do_group(g, gi):
        ib = g * (8 * L)
        ox, oy, oz = (inp_v[pl.ds(ib + c * L, L)] for c in (0, 1, 2))
        dx, dy, dz = (inp_v[pl.ds(ib + c * L, L)] for c in (3, 4, 5))
        near = inp_v[pl.ds(ib + 6 * L, L)]
        far = inp_v[pl.ds(ib + 7 * L, L)]
        fmn = far - near

        def sdf_at(x):
            t = near + x * fmn
            px = ox + dx * t
            py = oy + dy * t
            pz = oz + dz * t
            return _sqrtv(px * px + py * py + pz * pz) - 1.0

        def init_body(j, _):
            xv = jnp.broadcast_to(
                lax.convert_element_type(j, jnp.float32) * (1.0 / NSAMP), (L,))
            setrow(spb_a, j, xv)

            @pl.when(j < NSAMP)
            def _():
                setrow(sdf_a, j, sdf_at(xv))
            return 0

        lax.fori_loop(0, NSAMP + 1, init_body, 0)

        bufs = [(spb_a, sdf_a), (spb_b, sdf_b)]
        for it in range(NSTEP):
            S = NSAMP + NPER * it
            spb_src, sdf_src = bufs[it % 2]
            spb_dst, sdf_dst = bufs[(it + 1) % 2]
            last = it == NSTEP - 1
            inv_s = BASEVAR * (2.0 ** it)

            def passA(s, carry):
                trans, pcraw, ws, sdf_s, spb_s = carry
                sdf_n = row(sdf_src, s + 1)
                spb_n = row(spb_src, s + 1)
                dlt = (spb_n - spb_s) * fmn
                cos = (sdf_n - sdf_s) / (dlt + 1e-5)
                cv = jnp.clip(jnp.minimum(pcraw, cos), -1e3, 0.0)
                mid = (sdf_s + sdf_n) * 0.5
                pc = _sigmoid((mid - cv * dlt * 0.5) * inv_s)
                nc = _sigmoid((mid + cv * dlt * 0.5) * inv_s)
                alpha = (pc - nc + 1e-5) / (pc + 1e-5)
                wv = alpha * trans + 1e-5
                trans = trans * (1.0 - alpha + 1e-7)
                setrow(wbuf, s, wv)
                return (trans, cos, ws + wv, sdf_n, spb_n)

            carry0 = (jnp.ones((L,), jnp.float32), zf, zf,
                      row(sdf_src, 0), row(spb_src, 0))
            _, _, ws, _, _ = lax.fori_loop(0, S - 1, passA, carry0)
            setrow(wbuf, S - 1, jnp.full((L,), 1e-5, jnp.float32))
            ws = ws + 1e-5
            pad = jnp.maximum(0.0, 1e-5 - ws)
            padn = pad * (1.0 / S)
            inv_ws = 1.0 / (ws + pad)

            setrow(cdf, 0, zf)

            def passB(s, c):
                c = c + (row(wbuf, s) + padn) * inv_ws
                setrow(cdf, s + 1, jnp.minimum(1.0, c))
                return c

            lax.fori_loop(0, S, passB, zf)

            def pdfk(k, _):
                kf = lax.convert_element_type(k, jnp.float32)
                u = jnp.broadcast_to(kf * (1.0 / 17.0) + (1.0 / 34.0), (L,))
                inds = _search_right(cdf, zf, u, S + 1, lane)
                below = jnp.clip(inds - 1, 0, S) * L + lane
                above = jnp.clip(inds, 0, S) * L + lane
                g0 = plsc.load_gather(cdf, [below])
                g1 = plsc.load_gather(cdf, [above])
                b0 = plsc.load_gather(spb_src, [below])
                b1 = plsc.load_gather(spb_src, [above])
                t = (u - g0) / (g1 - g0)
                t = jnp.where(t != t, 0.0, t)
                t = jnp.clip(t, 0.0, 1.0)
                setrow(nbuf, k, b0 + t * (b1 - b0))
                return 0

            lax.fori_loop(0, NBINS, pdfk, 0)

            if not last:
                def sdfb_body(j, _):
                    setrow(sdfb, j, sdf_at(row(nbuf, j)))
                    return 0

                lax.fori_loop(0, NPER, sdfb_body, 0)

            def zero_body(i, _):
                setrow(delta, i, zi)
                return 0

            lax.fori_loop(0, S + 1, zero_body, 0)

            ob = gi * (NOUT * L)

            def posb_body(j, carry):
                bv = row(nbuf, j)
                pb = _search_right(spb_src, zf, bv, S, lane)
                plsc.addupdate_scatter(delta, [pb * L + lane], onei)
                rb = pb + jnp.broadcast_to(j, (L,)).astype(jnp.int32)
                if last:
                    plsc.store_scatter(outc_v, [ob + rb * L + lane],
                                       near + bv * fmn)
                else:
                    plsc.store_scatter(spb_dst, [rb * L + lane], bv)
                    plsc.store_scatter(sdf_dst, [rb * L + lane], row(sdfb, j))
                return carry

            lax.fori_loop(0, NPER, posb_body, 0)

            def apass(i, cnt):
                cnt = cnt + row(delta, i)
                r = cnt + jnp.broadcast_to(i, (L,)).astype(jnp.int32)
                av = row(spb_src, i)
                if last:
                    plsc.store_scatter(outc_v, [ob + r * L + lane],
                                       near + av * fmn)
                else:
                    plsc.store_scatter(spb_dst, [r * L + lane], av)
                    plsc.store_scatter(sdf_dst, [r * L + lane], row(sdf_src, i))
                return cnt

            lax.fori_loop(0, S, apass, zi)

            endv = jnp.maximum(row(spb_src, S), row(nbuf, NPER))
            if last:
                outc_v[pl.ds(ob + SOUT * L, L)] = near + endv * fmn
            else:
                setrow(spb_dst, S + NPER, endv)

    def chunk_body(c, _):
        def group_body(gg, _):
            do_group(c * CH + gg, gg)
            return 0

        lax.fori_loop(0, CH, group_body, 0)
        pltpu.sync_copy(
            outc_v,
            out_hbm.at[pl.ds((wid * GPW + c * CH) * (NOUT * L), CH * NOUT * L)])
        return 0

    lax.fori_loop(0, NCHUNK, chunk_body, 0)


_mesh = plsc.VectorSubcoreMesh(core_axis_name="c", subcore_axis_name="s")

_sc_sampler = functools.partial(
    pl.kernel,
    out_type=jax.ShapeDtypeStruct((GROUPS * NOUT * L,), jnp.float32),
    mesh=_mesh,
    compiler_params=pltpu.CompilerParams(needs_layout_passes=False),
    scratch_types=[
        pltpu.VMEM((GPW * 8 * L,), jnp.float32),
        pltpu.VMEM((CH * NOUT * L,), jnp.float32),
        pltpu.VMEM((NOUT * L,), jnp.float32),
        pltpu.VMEM((SOUT * L,), jnp.float32),
        pltpu.VMEM((NOUT * L,), jnp.float32),
        pltpu.VMEM((SOUT * L,), jnp.float32),
        pltpu.VMEM((SOUT * L,), jnp.float32),
        pltpu.VMEM((NOUT * L,), jnp.float32),
        pltpu.VMEM((NBINS * L,), jnp.float32),
        pltpu.VMEM((NPER * L,), jnp.float32),
        pltpu.VMEM((NOUT * L,), jnp.int32),
    ],
)(_sc_body)


def kernel(origins, directions, nears, fars):
    ox = origins[:, 0].reshape(GROUPS, L)
    oy = origins[:, 1].reshape(GROUPS, L)
    oz = origins[:, 2].reshape(GROUPS, L)
    dx = directions[:, 0].reshape(GROUPS, L)
    dy = directions[:, 1].reshape(GROUPS, L)
    dz = directions[:, 2].reshape(GROUPS, L)
    nr = nears[:, 0].reshape(GROUPS, L)
    fr = fars[:, 0].reshape(GROUPS, L)
    inp = jnp.stack([ox, oy, oz, dx, dy, dz, nr, fr], axis=1)
    out = _sc_sampler(inp.reshape(-1))
    return out.reshape(GROUPS, NOUT, L).transpose(0, 2, 1).reshape(NRAYS, NOUT)


# speedup vs baseline: 7.8299x; 1.1809x over previous
import functools

import jax
import jax.numpy as jnp
from jax import lax
from jax.experimental import pallas as pl
from jax.experimental.pallas import tpu as pltpu
from jax.experimental.pallas import tpu_sc as plsc

NSAMP = 64
NSTEP = 4
NPER = 16
NBINS = NPER + 1
BASEVAR = 64.0
NRAYS = 65536
L = 16
NC, NS = 2, 16
NWORK = NC * NS
GROUPS = NRAYS // L
GPW = GROUPS // NWORK
CH = 8
NCHUNK = GPW // CH
SOUT = NSAMP + NSTEP * NPER
NOUT = SOUT + 1


def _sqrtv(a):
    i = lax.bitcast_convert_type(a, jnp.int32)
    x = lax.bitcast_convert_type(jnp.int32(0x5F3759DF) - (i >> 1), jnp.float32)
    for _ in range(3):
        x = x * (1.5 - 0.5 * a * x * x)
    return a * x


def _sigmoid(z):
    return 1.0 / (1.0 + jnp.exp(-z))


def _search_right(ref, v, length, lane):
    pos = jnp.zeros((L,), jnp.int32)
    step = 64
    while step >= 1:
        cand = jnp.minimum(pos + step, length - 1)
        av = plsc.load_gather(ref, [cand * L + lane])
        pos = jnp.where(av <= v, cand, pos)
        step //= 2
    return pos + 1


def _sc_body(inp_hbm, out_hbm, inp_v, outc_v,
             spb_a, sdf_a, spb_b, sdf_b, cdf, nbuf, delta):
    cid = lax.axis_index("c")
    sid = lax.axis_index("s")
    wid = sid * NC + cid
    lane = lax.iota(jnp.int32, L)
    zf = jnp.zeros((L,), jnp.float32)
    zi = jnp.zeros((L,), jnp.int32)
    onei = jnp.ones((L,), jnp.int32)

    pltpu.sync_copy(inp_hbm.at[pl.ds(wid * (GPW * 8 * L), GPW * 8 * L)], inp_v)

    def row(ref, i):
        return ref[pl.ds(i * L, L)]

    def setrow(ref, i, v):
        ref[pl.ds(i * L, L)] = v

    def zero_body(i, _):
        setrow(delta, i, zi)
        return 0

    lax.fori_loop(0, NOUT, zero_body, 0)

    def do_group(g, gi):
        ib = g * (8 * L)
        ox, oy, oz = (inp_v[pl.ds(ib + c * L, L)] for c in (0, 1, 2))
        dx, dy, dz = (inp_v[pl.ds(ib + c * L, L)] for c in (3, 4, 5))
        near = inp_v[pl.ds(ib + 6 * L, L)]
        far = inp_v[pl.ds(ib + 7 * L, L)]
        fmn = far - near

        def sdf_at(x):
            t = near + x * fmn
            px = ox + dx * t
            py = oy + dy * t
            pz = oz + dz * t
            return _sqrtv(px * px + py * py + pz * pz) - 1.0

        def init_body(j, _):
            xv = jnp.broadcast_to(
                lax.convert_element_type(j, jnp.float32) * (1.0 / NSAMP), (L,))
            setrow(spb_a, j, xv)

            @pl.when(j < NSAMP)
            def _():
                setrow(sdf_a, j, sdf_at(xv))
            return 0

        lax.fori_loop(0, NSAMP + 1, init_body, 0)
        setrow(cdf, 0, zf)

        bufs = [(spb_a, sdf_a), (spb_b, sdf_b)]
        for it in range(NSTEP):
            S = NSAMP + NPER * it
            spb_src, sdf_src = bufs[it % 2]
            spb_dst, sdf_dst = bufs[(it + 1) % 2]
            last = it == NSTEP - 1
            inv_s = BASEVAR * (2.0 ** it)
            ob = gi * (NOUT * L)

            def passA(s, carry):
                trans, pcraw, c, sdf_s, spb_s = carry
                sdf_n = row(sdf_src, s + 1)
                spb_n = row(spb_src, s + 1)
                dlt = (spb_n - spb_s) * fmn
                cos = (sdf_n - sdf_s) / (dlt + 1e-5)
                cv = jnp.clip(jnp.minimum(pcraw, cos), -1e3, 0.0)
                mid = (sdf_s + sdf_n) * 0.5
                pc = _sigmoid((mid - cv * dlt * 0.5) * inv_s)
                nc = _sigmoid((mid + cv * dlt * 0.5) * inv_s)
                alpha = (pc - nc + 1e-5) / (pc + 1e-5)
                c = c + alpha * trans + 1e-5
                trans = trans * (1.0 - alpha + 1e-7)
                setrow(cdf, s + 1, c)
                return (trans, cos, c, sdf_n, spb_n)

            carry0 = (jnp.ones((L,), jnp.float32), zf, zf,
                      row(sdf_src, 0), row(spb_src, 0))
            _, _, c_end, _, _ = lax.fori_loop(0, S - 1, passA, carry0)
            ws = c_end + 1e-5
            setrow(cdf, S, ws)

            def sample_one(k, kf):
                u = jnp.broadcast_to(kf * (1.0 / 17.0) + (1.0 / 34.0), (L,)) * ws
                inds = _search_right(cdf, u, S + 1, lane)
                below = jnp.clip(inds - 1, 0, S) * L + lane
                above = jnp.clip(inds, 0, S) * L + lane
                g0 = plsc.load_gather(cdf, [below])
                g1 = plsc.load_gather(cdf, [above])
                b0 = plsc.load_gather(spb_src, [below])
                b1 = plsc.load_gather(spb_src, [above])
                t = (u - g0) / (g1 - g0)
                t = jnp.where(t != t, 0.0, t)
                t = jnp.clip(t, 0.0, 1.0)
                return b0 + t * (b1 - b0)

            def pdfk(k, _):
                bv = sample_one(k, lax.convert_element_type(k, jnp.float32))
                pb = _search_right(spb_src, bv, S, lane)
                plsc.addupdate_scatter(delta, [pb * L + lane], onei)
                rb = (pb + jnp.broadcast_to(k, (L,)).astype(jnp.int32)) * L + lane
                if last:
                    plsc.store_scatter(outc_v, [ob + rb], near + bv * fmn)
                else:
                    plsc.store_scatter(spb_dst, [rb], bv)
                    plsc.store_scatter(sdf_dst, [rb], sdf_at(bv))
                return 0

            lax.fori_loop(0, NPER, pdfk, 0)
            end_b = sample_one(NPER, jnp.float32(NPER))

            def apass(i, cnt):
                dv = row(delta, i)
                setrow(delta, i, zi)
                cnt = cnt + dv
                r = (cnt + jnp.broadcast_to(i, (L,)).astype(jnp.int32)) * L + lane
                av = row(spb_src, i)
                if last:
                    plsc.store_scatter(outc_v, [ob + r], near + av * fmn)
                else:
                    plsc.store_scatter(spb_dst, [r], av)
                    plsc.store_scatter(sdf_dst, [r], row(sdf_src, i))
                return cnt

            lax.fori_loop(0, S, apass, zi)
            setrow(delta, S, zi)

            endv = jnp.maximum(row(spb_src, S), end_b)
            if last:
                outc_v[pl.ds(ob + SOUT * L, L)] = near + endv * fmn
            else:
                setrow(spb_dst, S + NPER, endv)

    def chunk_body(c, _):
        def group_body(gg, _):
            do_group(c * CH + gg, gg)
            return 0

        lax.fori_loop(0, CH, group_body, 0)
        pltpu.sync_copy(
            outc_v,
            out_hbm.at[pl.ds((wid * GPW + c * CH) * (NOUT * L), CH * NOUT * L)])
        return 0

    lax.fori_loop(0, NCHUNK, chunk_body, 0)


_mesh = plsc.VectorSubcoreMesh(core_axis_name="c", subcore_axis_name="s")

_sc_sampler = functools.partial(
    pl.kernel,
    out_type=jax.ShapeDtypeStruct((GROUPS * NOUT * L,), jnp.float32),
    mesh=_mesh,
    compiler_params=pltpu.CompilerParams(needs_layout_passes=False),
    scratch_types=[
        pltpu.VMEM((GPW * 8 * L,), jnp.float32),
        pltpu.VMEM((CH * NOUT * L,), jnp.float32),
        pltpu.VMEM((NOUT * L,), jnp.float32),
        pltpu.VMEM((SOUT * L,), jnp.float32),
        pltpu.VMEM((NOUT * L,), jnp.float32),
        pltpu.VMEM((SOUT * L,), jnp.float32),
        pltpu.VMEM((NOUT * L,), jnp.float32),
        pltpu.VMEM((NBINS * L,), jnp.float32),
        pltpu.VMEM((NOUT * L,), jnp.int32),
    ],
)(_sc_body)


def kernel(origins, directions, nears, fars):
    ox = origins[:, 0].reshape(GROUPS, L)
    oy = origins[:, 1].reshape(GROUPS, L)
    oz = origins[:, 2].reshape(GROUPS, L)
    dx = directions[:, 0].reshape(GROUPS, L)
    dy = directions[:, 1].reshape(GROUPS, L)
    dz = directions[:, 2].reshape(GROUPS, L)
    nr = nears[:, 0].reshape(GROUPS, L)
    fr = fars[:, 0].reshape(GROUPS, L)
    inp = jnp.stack([ox, oy, oz, dx, dy, dz, nr, fr], axis=1)
    out = _sc_sampler(inp.reshape(-1))
    return out.reshape(GROUPS, NOUT, L).transpose(0, 2, 1).reshape(NRAYS, NOUT)
